# Initial kernel scaffold; baseline (speedup 1.0000x reference)
import functools

import jax
import jax.numpy as jnp
from jax import lax
from jax.experimental import pallas as pl
from jax.experimental.pallas import tpu as pltpu
from jax.experimental.pallas import tpu_sc as plsc

NU = 50000
NI = 50000
NN = NU + NI
EDG = 1600000
K = 32
KH = 16
NL = 3

NC = 2
NS = 16

CH = 4000
PER_TEC = EDG // NS
N_CH = PER_TEC // CH
ROWS_PER_TEC = NN // NS

CH_D = 5000
PER_TEC_D = EDG // (NC * NS)
N_CH_D = PER_TEC_D // CH_D

_mesh = plsc.VectorSubcoreMesh(core_axis_name="c", subcore_axis_name="s")


@functools.partial(
    pl.kernel,
    out_type=jax.ShapeDtypeStruct((NC, NN, KH), jnp.float32),
    mesh=_mesh,
    scratch_types=[
        pltpu.VMEM((CH,), jnp.int32),
        pltpu.VMEM((CH,), jnp.int32),
        pltpu.VMEM((CH, KH), jnp.float32),
        pltpu.VMEM_SHARED((NN, KH), jnp.float32),
        pltpu.SemaphoreType.DMA,
    ],
)
def _sc_spmm(xcat, col2, rowi, zeros, out, colv, rowv, gat, acc, sem):
    c = lax.axis_index("c")
    s = lax.axis_index("s")
    pltpu.sync_copy(zeros.at[pl.ds(s * ROWS_PER_TEC, ROWS_PER_TEC)],
                    acc.at[pl.ds(s * ROWS_PER_TEC, ROWS_PER_TEC)])
    plsc.subcore_barrier()

    def body(g, carry):
        base = s * PER_TEC + g * CH
        pltpu.sync_copy(col2.at[c, pl.ds(base, CH)], colv)
        pltpu.sync_copy(rowi.at[pl.ds(base, CH)], rowv)
        pltpu.async_copy(xcat.at[colv], gat, sem).wait()
        pltpu.sync_copy(gat, acc.at[rowv], add=True)
        return carry

    lax.fori_loop(0, N_CH, body, 0)
    plsc.subcore_barrier()
    pltpu.sync_copy(acc.at[pl.ds(s * ROWS_PER_TEC, ROWS_PER_TEC)],
                    out.at[c, pl.ds(s * ROWS_PER_TEC, ROWS_PER_TEC)])


@functools.partial(
    pl.kernel,
    out_type=jax.ShapeDtypeStruct((NC, NN, KH), jnp.float32),
    mesh=_mesh,
    scratch_types=[
        pltpu.VMEM((CH_D,), jnp.int32),
        pltpu.VMEM((CH_D, KH), jnp.float32),
        pltpu.VMEM_SHARED((NN, KH), jnp.float32),
    ],
)
def _sc_deg(rowi, ones, zeros, out, rowv, onev, acc):
    c = lax.axis_index("c")
    s = lax.axis_index("s")
    pltpu.sync_copy(zeros.at[pl.ds(s * ROWS_PER_TEC, ROWS_PER_TEC)],
                    acc.at[pl.ds(s * ROWS_PER_TEC, ROWS_PER_TEC)])
    pltpu.sync_copy(ones, onev)
    plsc.subcore_barrier()

    def body(g, carry):
        base = (c * NS + s) * PER_TEC_D + g * CH_D
        pltpu.sync_copy(rowi.at[pl.ds(base, CH_D)], rowv)
        pltpu.sync_copy(onev, acc.at[rowv], add=True)
        return carry

    lax.fori_loop(0, N_CH_D, body, 0)
    plsc.subcore_barrier()
    pltpu.sync_copy(acc.at[pl.ds(s * ROWS_PER_TEC, ROWS_PER_TEC)],
                    out.at[c, pl.ds(s * ROWS_PER_TEC, ROWS_PER_TEC)])


BLK = 2000
GRID = NN // BLK


def _tc_body(xh, agg, degp, filt, out):
    x = jnp.concatenate([xh[0], xh[1]], axis=1)
    a = jnp.concatenate([agg[0], agg[1]], axis=1)
    deg = degp[0, :, 0:1] + degp[1, :, 0:1] + 1e-7
    h = 2.0 * x - a / deg
    y = jax.nn.sigmoid(jnp.dot(h, filt[...], preferred_element_type=jnp.float32))
    out[0] = y[:, :KH]
    out[1] = y[:, KH:]


def _tc_dense(xh, agg, degp, filt):
    spec3 = pl.BlockSpec((NC, BLK, KH), lambda i: (0, i, 0))
    return pl.pallas_call(
        _tc_body,
        out_shape=jax.ShapeDtypeStruct((NC, NN, KH), jnp.float32),
        grid=(GRID,),
        in_specs=[spec3, spec3, spec3, pl.BlockSpec((K, K), lambda i: (0, 0))],
        out_specs=spec3,
    )(xh, agg, degp, filt)


def kernel(Gu, Gi, filters, edge_index):
    rowi = edge_index[0]
    coli = edge_index[1]
    col2 = jnp.stack([coli, coli + NN])
    x0 = jnp.concatenate([Gu, Gi], axis=0)
    xh = jnp.stack([x0[:, :KH], x0[:, KH:]])
    zeros = jnp.zeros((NN, KH), jnp.float32)
    ones = jnp.ones((CH_D, KH), jnp.float32)

    degp = _sc_deg(rowi, ones, zeros)

    outs = [x0]
    h = xh
    for k in range(NL):
        agg = _sc_spmm(h.reshape(NC * NN, KH), col2, rowi, zeros)
        h = _tc_dense(h, agg, degp, filters[k])
        outs.append(jnp.concatenate([h[0], h[1]], axis=1))

    emb = jnp.concatenate(outs, axis=1)
    return emb[:NU], emb[NU:]


# speedup vs baseline: 15.3711x; 15.3711x over previous
import functools

import jax
import jax.numpy as jnp
from jax import lax
from jax.experimental import pallas as pl
from jax.experimental.pallas import tpu as pltpu
from jax.experimental.pallas import tpu_sc as plsc

NU = 50000
NI = 50000
NN = NU + NI
NNP = 100096
EDG = 1600000
K = 32
KH = 16
NL = 3

NC = 2
NS = 16

CH = 1000
PER_TEC = EDG // NS
N_CH = PER_TEC // CH
ROWS_PER_TEC = NNP // NS

CH_D = 1000
PER_TEC_D = EDG // (NC * NS)
N_CH_D = PER_TEC_D // CH_D

_mesh = plsc.VectorSubcoreMesh(core_axis_name="c", subcore_axis_name="s")


@functools.partial(
    pl.kernel,
    out_type=jax.ShapeDtypeStruct((NC, NNP, KH), jnp.float32),
    mesh=_mesh,
    scratch_types=[
        pltpu.VMEM((CH,), jnp.int32),
        pltpu.VMEM((CH,), jnp.int32),
        pltpu.VMEM((CH, KH), jnp.float32),
        pltpu.VMEM_SHARED((NNP, KH), jnp.float32),
        pltpu.SemaphoreType.DMA,
    ],
    compiler_params=pltpu.CompilerParams(use_tc_tiling_on_sc=False),
)
def _sc_spmm(xcat, col2, rowi, zeros, out, colv, rowv, gat, acc, sem):
    c = lax.axis_index("c")
    s = lax.axis_index("s")
    pltpu.sync_copy(zeros.at[pl.ds(s * ROWS_PER_TEC, ROWS_PER_TEC)],
                    acc.at[pl.ds(s * ROWS_PER_TEC, ROWS_PER_TEC)])
    plsc.subcore_barrier()

    def body(g, carry):
        base = s * PER_TEC + g * CH
        pltpu.sync_copy(col2.at[pl.ds(c * EDG + base, CH)], colv)
        pltpu.sync_copy(rowi.at[pl.ds(base, CH)], rowv)
        pltpu.async_copy(xcat.at[colv], gat, sem).wait()
        pltpu.sync_copy(gat, acc.at[rowv], add=True)
        return carry

    lax.fori_loop(0, N_CH, body, 0)
    plsc.subcore_barrier()
    pltpu.sync_copy(acc.at[pl.ds(s * ROWS_PER_TEC, ROWS_PER_TEC)],
                    out.at[c, pl.ds(s * ROWS_PER_TEC, ROWS_PER_TEC)])


@functools.partial(
    pl.kernel,
    out_type=jax.ShapeDtypeStruct((NC, NNP, KH), jnp.float32),
    mesh=_mesh,
    scratch_types=[
        pltpu.VMEM((CH_D,), jnp.int32),
        pltpu.VMEM((CH_D, KH), jnp.float32),
        pltpu.VMEM_SHARED((NNP, KH), jnp.float32),
    ],
    compiler_params=pltpu.CompilerParams(use_tc_tiling_on_sc=False),
)
def _sc_deg(rowi, ones, zeros, out, rowv, onev, acc):
    c = lax.axis_index("c")
    s = lax.axis_index("s")
    pltpu.sync_copy(zeros.at[pl.ds(s * ROWS_PER_TEC, ROWS_PER_TEC)],
                    acc.at[pl.ds(s * ROWS_PER_TEC, ROWS_PER_TEC)])
    pltpu.sync_copy(ones, onev)
    plsc.subcore_barrier()

    def body(g, carry):
        base = (c * NS + s) * PER_TEC_D + g * CH_D
        pltpu.sync_copy(rowi.at[pl.ds(base, CH_D)], rowv)
        pltpu.sync_copy(onev, acc.at[rowv], add=True)
        return carry

    lax.fori_loop(0, N_CH_D, body, 0)
    plsc.subcore_barrier()
    pltpu.sync_copy(acc.at[pl.ds(s * ROWS_PER_TEC, ROWS_PER_TEC)],
                    out.at[c, pl.ds(s * ROWS_PER_TEC, ROWS_PER_TEC)])


BLK = 6256
GRID = NNP // BLK


def _tc_body(xh, agg, degp, filt, out):
    x = jnp.concatenate([xh[0], xh[1]], axis=1)
    a = jnp.concatenate([agg[0], agg[1]], axis=1)
    deg = degp[0, :, 0:1] + degp[1, :, 0:1] + 1e-7
    h = 2.0 * x - a / deg
    y = jax.nn.sigmoid(jnp.dot(h, filt[...], preferred_element_type=jnp.float32))
    out[0] = y[:, :KH]
    out[1] = y[:, KH:]


def _tc_dense(xh, agg, degp, filt):
    spec3 = pl.BlockSpec((NC, BLK, KH), lambda i: (0, i, 0))
    return pl.pallas_call(
        _tc_body,
        out_shape=jax.ShapeDtypeStruct((NC, NNP, KH), jnp.float32),
        grid=(GRID,),
        in_specs=[spec3, spec3, spec3, pl.BlockSpec((K, K), lambda i: (0, 0))],
        out_specs=spec3,
    )(xh, agg, degp, filt)


def kernel(Gu, Gi, filters, edge_index):
    rowi = edge_index[0]
    coli = edge_index[1]
    col2 = jnp.concatenate([coli, coli + NNP])
    x0 = jnp.concatenate(
        [Gu, Gi, jnp.zeros((NNP - NN, K), jnp.float32)], axis=0)
    xh = jnp.stack([x0[:, :KH], x0[:, KH:]])
    zeros = jnp.zeros((NNP, KH), jnp.float32)
    ones = jnp.ones((CH_D, KH), jnp.float32)

    degp = _sc_deg(rowi, ones, zeros)

    outs = [x0]
    h = xh
    for k in range(NL):
        agg = _sc_spmm(h.reshape(NC * NNP, KH), col2, rowi, zeros)
        h = _tc_dense(h, agg, degp, filters[k])
        outs.append(jnp.concatenate([h[0], h[1]], axis=1))

    emb = jnp.concatenate(outs, axis=1)
    return emb[:NU], emb[NU:NN]


# speedup vs baseline: 21.2599x; 1.3831x over previous
import functools

import jax
import jax.numpy as jnp
from jax import lax
from jax.experimental import pallas as pl
from jax.experimental.pallas import tpu as pltpu
from jax.experimental.pallas import tpu_sc as plsc

NU = 50000
NI = 50000
NN = NU + NI
NNP = 102400
P = NNP // 8
EDG = 1600000
K = 32
KH = 16
NL = 3

NC = 2
NS = 16

CH = 1000
PER_TEC = EDG // NS
N_CH = PER_TEC // CH
ROWS_PER_TEC = NNP // NS

CH_D = 1000
PER_TEC_D = EDG // (NC * NS)
N_CH_D = PER_TEC_D // CH_D

_mesh = plsc.VectorSubcoreMesh(core_axis_name="c", subcore_axis_name="s")


@functools.partial(
    pl.kernel,
    out_type=jax.ShapeDtypeStruct((NC, NNP, KH), jnp.float32),
    mesh=_mesh,
    scratch_types=[
        pltpu.VMEM((CH,), jnp.int32),
        pltpu.VMEM((CH,), jnp.int32),
        pltpu.VMEM((CH, KH), jnp.float32),
        pltpu.VMEM_SHARED((NNP, KH), jnp.float32),
        pltpu.SemaphoreType.DMA,
    ],
    compiler_params=pltpu.CompilerParams(use_tc_tiling_on_sc=False),
)
def _sc_spmm(xcat, col2, rowi, zeros, out, colv, rowv, gat, acc, sem):
    c = lax.axis_index("c")
    s = lax.axis_index("s")
    pltpu.sync_copy(zeros.at[pl.ds(s * ROWS_PER_TEC, ROWS_PER_TEC)],
                    acc.at[pl.ds(s * ROWS_PER_TEC, ROWS_PER_TEC)])
    plsc.subcore_barrier()

    def body(g, carry):
        base = s * PER_TEC + g * CH
        pltpu.sync_copy(col2.at[pl.ds(c * EDG + base, CH)], colv)
        pltpu.sync_copy(rowi.at[pl.ds(base, CH)], rowv)
        pltpu.async_copy(xcat.at[colv], gat, sem).wait()
        pltpu.sync_copy(gat, acc.at[rowv], add=True)
        return carry

    lax.fori_loop(0, N_CH, body, 0)
    plsc.subcore_barrier()
    pltpu.sync_copy(acc.at[pl.ds(s * ROWS_PER_TEC, ROWS_PER_TEC)],
                    out.at[c, pl.ds(s * ROWS_PER_TEC, ROWS_PER_TEC)])


@functools.partial(
    pl.kernel,
    out_type=jax.ShapeDtypeStruct((NC, NNP, KH), jnp.float32),
    mesh=_mesh,
    scratch_types=[
        pltpu.VMEM((CH_D,), jnp.int32),
        pltpu.VMEM((CH_D, KH), jnp.float32),
        pltpu.VMEM_SHARED((NNP, KH), jnp.float32),
    ],
    compiler_params=pltpu.CompilerParams(use_tc_tiling_on_sc=False),
)
def _sc_deg(rowi, ones, zeros, out, rowv, onev, acc):
    c = lax.axis_index("c")
    s = lax.axis_index("s")
    pltpu.sync_copy(zeros.at[pl.ds(s * ROWS_PER_TEC, ROWS_PER_TEC)],
                    acc.at[pl.ds(s * ROWS_PER_TEC, ROWS_PER_TEC)])
    pltpu.sync_copy(ones, onev)
    plsc.subcore_barrier()

    def body(g, carry):
        base = (c * NS + s) * PER_TEC_D + g * CH_D
        pltpu.sync_copy(rowi.at[pl.ds(base, CH_D)], rowv)
        pltpu.sync_copy(onev, acc.at[rowv], add=True)
        return carry

    lax.fori_loop(0, N_CH_D, body, 0)
    plsc.subcore_barrier()
    pltpu.sync_copy(acc.at[pl.ds(s * ROWS_PER_TEC, ROWS_PER_TEC)],
                    out.at[c, pl.ds(s * ROWS_PER_TEC, ROWS_PER_TEC)])


BLK_P = 1600
GRID = P // BLK_P


def _tc_body(xh, agg, degp, bd, out):
    invd = 1.0 / (degp[0] + degp[1] + 1e-7)
    hl = 2.0 * xh[0] - agg[0] * invd
    hr = 2.0 * xh[1] - agg[1] * invd
    yl = jax.nn.sigmoid(
        jnp.dot(hl, bd[0, 0], preferred_element_type=jnp.float32)
        + jnp.dot(hr, bd[1, 0], preferred_element_type=jnp.float32))
    yr = jax.nn.sigmoid(
        jnp.dot(hl, bd[0, 1], preferred_element_type=jnp.float32)
        + jnp.dot(hr, bd[1, 1], preferred_element_type=jnp.float32))
    out[0] = yl
    out[1] = yr


def _tc_dense(xh, agg, degp, bd):
    spec3 = pl.BlockSpec((NC, BLK_P, 128), lambda i: (0, i, 0))
    return pl.pallas_call(
        _tc_body,
        out_shape=jax.ShapeDtypeStruct((NC, P, 128), jnp.float32),
        grid=(GRID,),
        in_specs=[spec3, spec3, spec3,
                  pl.BlockSpec((2, 2, 128, 128), lambda i: (0, 0, 0, 0))],
        out_specs=spec3,
    )(xh, agg, degp, bd)


def kernel(Gu, Gi, filters, edge_index):
    rowi = edge_index[0]
    coli = edge_index[1]
    col2 = jnp.concatenate([coli, coli + NNP])
    x0 = jnp.concatenate(
        [Gu, Gi, jnp.zeros((NNP - NN, K), jnp.float32)], axis=0)
    yp = jnp.stack([x0[:, :KH].reshape(P, 128),
                    x0[:, KH:].reshape(P, 128)])
    zeros = jnp.zeros((NNP, KH), jnp.float32)
    ones = jnp.ones((CH_D, KH), jnp.float32)

    eye8 = jnp.eye(8, dtype=jnp.float32)
    bds = jnp.stack([
        jnp.stack([
            jnp.stack([jnp.kron(eye8, filters[k][:KH, :KH]),
                       jnp.kron(eye8, filters[k][:KH, KH:])]),
            jnp.stack([jnp.kron(eye8, filters[k][KH:, :KH]),
                       jnp.kron(eye8, filters[k][KH:, KH:])]),
        ]) for k in range(NL)])

    degp = _sc_deg(rowi, ones, zeros).reshape(NC, P, 128)

    pieces = [yp]
    for k in range(NL):
        agg = _sc_spmm(yp.reshape(NC * NNP, KH), col2, rowi,
                       zeros).reshape(NC, P, 128)
        yp = _tc_dense(yp, agg, degp, bds[k])
        pieces.append(yp)

    emb = jnp.concatenate(
        [p[h].reshape(NNP, KH) for p in pieces for h in (0, 1)], axis=1)
    return emb[:NU], emb[NU:NN]


# speedup vs baseline: 29.0389x; 1.3659x over previous
import functools

import jax
import jax.numpy as jnp
from jax import lax
from jax.experimental import pallas as pl
from jax.experimental.pallas import tpu as pltpu
from jax.experimental.pallas import tpu_sc as plsc

NU = 50000
NI = 50000
NN = NU + NI
NNP = 100096
P = NNP // 8
EDG = 1600000
K = 32
KH = 16
NL = 3

NC = 2
NS = 16

CH = 800
PER_TEC = EDG // NS
N_CH = PER_TEC // CH
ROWS_PER_TEC = NNP // NS

CH_D = 1000
PER_TEC_D = EDG // (NC * NS)
N_CH_D = PER_TEC_D // CH_D

_mesh = plsc.VectorSubcoreMesh(core_axis_name="c", subcore_axis_name="s")


@functools.partial(
    pl.kernel,
    out_type=jax.ShapeDtypeStruct((NC, NNP, KH), jnp.float32),
    mesh=_mesh,
    scratch_types=[
        pltpu.VMEM((2, CH), jnp.int32),
        pltpu.VMEM((2, CH), jnp.int32),
        pltpu.VMEM((2, CH, KH), jnp.float32),
        pltpu.VMEM_SHARED((NNP, KH), jnp.float32),
        pltpu.SemaphoreType.DMA,
        pltpu.SemaphoreType.DMA,
    ],
    compiler_params=pltpu.CompilerParams(use_tc_tiling_on_sc=False),
)
def _sc_spmm(xcat, col2, rowi, zeros, out, colv, rowv, gat, acc, semi, semg):
    c = lax.axis_index("c")
    s = lax.axis_index("s")
    pltpu.sync_copy(zeros.at[pl.ds(s * ROWS_PER_TEC, ROWS_PER_TEC)],
                    acc.at[pl.ds(s * ROWS_PER_TEC, ROWS_PER_TEC)])
    plsc.subcore_barrier()

    cbase = c * EDG + s * PER_TEC
    rbase = s * PER_TEC

    def start_idx(g, b):
        pltpu.async_copy(col2.at[pl.ds(cbase + g * CH, CH)], colv.at[b], semi)
        pltpu.async_copy(rowi.at[pl.ds(rbase + g * CH, CH)], rowv.at[b], semi)

    def wait_idx(b):
        pltpu.make_async_copy(col2.at[pl.ds(cbase, CH)], colv.at[b], semi).wait()
        pltpu.make_async_copy(rowi.at[pl.ds(rbase, CH)], rowv.at[b], semi).wait()

    def start_gather(b):
        pltpu.async_copy(xcat.at[colv.at[b]], gat.at[b], semg)

    def wait_gather(b):
        pltpu.make_async_copy(xcat.at[colv.at[b]], gat.at[b], semg).wait()

    start_idx(0, 0)
    wait_idx(0)
    start_idx(1, 1)
    start_gather(0)

    def body(g, carry):
        p = lax.rem(g, 2)
        q = 1 - p
        wait_gather(p)

        @pl.when(g + 1 < N_CH)
        def _():
            wait_idx(q)
            start_gather(q)

        pltpu.sync_copy(gat.at[p], acc.at[rowv.at[p]], add=True)

        @pl.when(g + 2 < N_CH)
        def _():
            start_idx(g + 2, p)

        return carry

    lax.fori_loop(0, N_CH, body, 0)
    plsc.subcore_barrier()
    pltpu.sync_copy(acc.at[pl.ds(s * ROWS_PER_TEC, ROWS_PER_TEC)],
                    out.at[c, pl.ds(s * ROWS_PER_TEC, ROWS_PER_TEC)])


@functools.partial(
    pl.kernel,
    out_type=jax.ShapeDtypeStruct((NC, NNP, KH), jnp.float32),
    mesh=_mesh,
    scratch_types=[
        pltpu.VMEM((CH_D,), jnp.int32),
        pltpu.VMEM((CH_D, KH), jnp.float32),
        pltpu.VMEM_SHARED((NNP, KH), jnp.float32),
    ],
    compiler_params=pltpu.CompilerParams(use_tc_tiling_on_sc=False),
)
def _sc_deg(rowi, ones, zeros, out, rowv, onev, acc):
    c = lax.axis_index("c")
    s = lax.axis_index("s")
    pltpu.sync_copy(zeros.at[pl.ds(s * ROWS_PER_TEC, ROWS_PER_TEC)],
                    acc.at[pl.ds(s * ROWS_PER_TEC, ROWS_PER_TEC)])
    pltpu.sync_copy(ones, onev)
    plsc.subcore_barrier()

    def body(g, carry):
        base = (c * NS + s) * PER_TEC_D + g * CH_D
        pltpu.sync_copy(rowi.at[pl.ds(base, CH_D)], rowv)
        pltpu.sync_copy(onev, acc.at[rowv], add=True)
        return carry

    lax.fori_loop(0, N_CH_D, body, 0)
    plsc.subcore_barrier()
    pltpu.sync_copy(acc.at[pl.ds(s * ROWS_PER_TEC, ROWS_PER_TEC)],
                    out.at[c, pl.ds(s * ROWS_PER_TEC, ROWS_PER_TEC)])


BLK_P = 3128
GRID = P // BLK_P


def _tc_body(xh, agg, degp, bd, out):
    invd = 1.0 / (degp[0] + degp[1] + 1e-7)
    hl = 2.0 * xh[0] - agg[0] * invd
    hr = 2.0 * xh[1] - agg[1] * invd
    yl = jax.nn.sigmoid(
        jnp.dot(hl, bd[0, 0], preferred_element_type=jnp.float32)
        + jnp.dot(hr, bd[1, 0], preferred_element_type=jnp.float32))
    yr = jax.nn.sigmoid(
        jnp.dot(hl, bd[0, 1], preferred_element_type=jnp.float32)
        + jnp.dot(hr, bd[1, 1], preferred_element_type=jnp.float32))
    out[0] = yl
    out[1] = yr


def _tc_dense(xh, agg, degp, bd):
    spec3 = pl.BlockSpec((NC, BLK_P, 128), lambda i: (0, i, 0))
    return pl.pallas_call(
        _tc_body,
        out_shape=jax.ShapeDtypeStruct((NC, P, 128), jnp.float32),
        grid=(GRID,),
        in_specs=[spec3, spec3, spec3,
                  pl.BlockSpec((2, 2, 128, 128), lambda i: (0, 0, 0, 0))],
        out_specs=spec3,
    )(xh, agg, degp, bd)


def kernel(Gu, Gi, filters, edge_index):
    rowi = edge_index[0]
    coli = edge_index[1]
    col2 = jnp.concatenate([coli, coli + NNP])
    x0 = jnp.concatenate(
        [Gu, Gi, jnp.zeros((NNP - NN, K), jnp.float32)], axis=0)
    yp = jnp.stack([x0[:, :KH].reshape(P, 128),
                    x0[:, KH:].reshape(P, 128)])
    zeros = jnp.zeros((NNP, KH), jnp.float32)
    ones = jnp.ones((CH_D, KH), jnp.float32)

    eye8 = jnp.eye(8, dtype=jnp.float32)
    bds = jnp.stack([
        jnp.stack([
            jnp.stack([jnp.kron(eye8, filters[k][:KH, :KH]),
                       jnp.kron(eye8, filters[k][:KH, KH:])]),
            jnp.stack([jnp.kron(eye8, filters[k][KH:, :KH]),
                       jnp.kron(eye8, filters[k][KH:, KH:])]),
        ]) for k in range(NL)])

    degp = _sc_deg(rowi, ones, zeros).reshape(NC, P, 128)

    pieces = [yp]
    for k in range(NL):
        agg = _sc_spmm(yp.reshape(NC * NNP, KH), col2, rowi,
                       zeros).reshape(NC, P, 128)
        yp = _tc_dense(yp, agg, degp, bds[k])
        pieces.append(yp)

    emb = jnp.concatenate(
        [p[h].reshape(NNP, KH) for p in pieces for h in (0, 1)], axis=1)
    return emb[:NU], emb[NU:NN]


# speedup vs baseline: 29.4518x; 1.0142x over previous
import functools

import jax
import jax.numpy as jnp
from jax import lax
from jax.experimental import pallas as pl
from jax.experimental.pallas import tpu as pltpu
from jax.experimental.pallas import tpu_sc as plsc

NU = 50000
NI = 50000
NN = NU + NI
NNP = 100096
P = NNP // 8
EDG = 1600000
K = 32
KH = 16
NL = 3

NC = 2
NS = 16

CH = 800
PER_TEC = EDG // NS
N_CH = PER_TEC // CH
ROWS_PER_TEC = NNP // NS

CH_D = 1000
PER_TEC_D = EDG // (NC * NS)
N_CH_D = PER_TEC_D // CH_D

_mesh = plsc.VectorSubcoreMesh(core_axis_name="c", subcore_axis_name="s")


@functools.partial(
    pl.kernel,
    out_type=jax.ShapeDtypeStruct((NC, NNP, KH), jnp.float32),
    mesh=_mesh,
    scratch_types=[
        pltpu.VMEM((3, CH), jnp.int32),
        pltpu.VMEM((3, CH), jnp.int32),
        pltpu.VMEM((2, CH, KH), jnp.float32),
        pltpu.VMEM_SHARED((NNP, KH), jnp.float32),
        pltpu.SemaphoreType.DMA,
        pltpu.SemaphoreType.DMA,
        pltpu.SemaphoreType.DMA,
    ],
    compiler_params=pltpu.CompilerParams(use_tc_tiling_on_sc=False),
)
def _sc_spmm(xcat, coli, rowi, out, colv, rowv, gat, acc, semi, semg, sems):
    c = lax.axis_index("c")
    s = lax.axis_index("s")
    slab = pl.ds(s * ROWS_PER_TEC, ROWS_PER_TEC)

    def fill_zero(j, carry):
        gat[0, j, :] = jnp.zeros((16,), jnp.float32)
        return carry

    lax.fori_loop(0, CH, fill_zero, 0)
    for r in range(7):
        pltpu.sync_copy(
            gat.at[0],
            acc.at[pl.ds(s * ROWS_PER_TEC + r * CH, CH)])
    pltpu.sync_copy(
        gat.at[0, pl.ds(0, 656)],
        acc.at[pl.ds(s * ROWS_PER_TEC + 7 * CH, 656)])
    plsc.subcore_barrier()

    base0 = s * PER_TEC

    def start_idx(g, b):
        pltpu.async_copy(coli.at[pl.ds(base0 + g * CH, CH)], colv.at[b], semi)
        pltpu.async_copy(rowi.at[pl.ds(base0 + g * CH, CH)], rowv.at[b], semi)

    def wait_idx(b):
        pltpu.make_async_copy(coli.at[pl.ds(base0, CH)], colv.at[b], semi).wait()
        pltpu.make_async_copy(rowi.at[pl.ds(base0, CH)], rowv.at[b], semi).wait()

    def start_gather(b3, b2):
        pltpu.async_copy(xcat.at[c].at[colv.at[b3]], gat.at[b2], semg)

    def wait_gather(b3, b2):
        pltpu.make_async_copy(xcat.at[c].at[colv.at[b3]], gat.at[b2], semg).wait()

    def start_scatter(b2, b3):
        pltpu.async_copy(gat.at[b2], acc.at[rowv.at[b3]], sems, add=True)

    def wait_scatter(b2, b3):
        pltpu.make_async_copy(gat.at[b2], acc.at[rowv.at[b3]], sems).wait()

    start_idx(0, 0)
    wait_idx(0)
    start_idx(1, 1)
    start_gather(0, 0)

    def body(g, carry):
        pg = lax.rem(g, 2)
        qg = 1 - pg
        b0 = lax.rem(g, 3)
        b1 = lax.rem(g + 1, 3)
        b2 = lax.rem(g + 2, 3)
        wait_gather(b0, pg)

        @pl.when(g >= 1)
        def _():
            wait_scatter(qg, lax.rem(g + 2, 3))

        @pl.when(g + 1 < N_CH)
        def _():
            wait_idx(b1)
            start_gather(b1, qg)

        start_scatter(pg, b0)

        @pl.when(g + 2 < N_CH)
        def _():
            start_idx(g + 2, b2)

        return carry

    lax.fori_loop(0, N_CH, body, 0)
    wait_scatter(lax.rem(N_CH - 1, 2), lax.rem(N_CH - 1, 3))
    plsc.subcore_barrier()
    pltpu.sync_copy(acc.at[slab], out.at[c, slab])


@functools.partial(
    pl.kernel,
    out_type=jax.ShapeDtypeStruct((NC, NNP, KH), jnp.float32),
    mesh=_mesh,
    scratch_types=[
        pltpu.VMEM((CH_D,), jnp.int32),
        pltpu.VMEM((CH_D, KH), jnp.float32),
        pltpu.VMEM_SHARED((NNP, KH), jnp.float32),
    ],
    compiler_params=pltpu.CompilerParams(use_tc_tiling_on_sc=False),
)
def _sc_deg(rowi, out, rowv, onev, acc):
    c = lax.axis_index("c")
    s = lax.axis_index("s")
    slab = pl.ds(s * ROWS_PER_TEC, ROWS_PER_TEC)

    def fill(j, val):
        onev[j, :] = jnp.full((16,), val, jnp.float32)
        return val

    lax.fori_loop(0, CH_D, lambda j, v: fill(j, 0.0), 0.0)
    for r in range(6):
        pltpu.sync_copy(
            onev, acc.at[pl.ds(s * ROWS_PER_TEC + r * CH_D, CH_D)])
    pltpu.sync_copy(
        onev.at[pl.ds(0, 256)],
        acc.at[pl.ds(s * ROWS_PER_TEC + 6 * CH_D, 256)])
    lax.fori_loop(0, CH_D, lambda j, v: fill(j, 1.0), 1.0)
    plsc.subcore_barrier()

    def body(g, carry):
        base = (c * NS + s) * PER_TEC_D + g * CH_D
        pltpu.sync_copy(rowi.at[pl.ds(base, CH_D)], rowv)
        pltpu.sync_copy(onev, acc.at[rowv], add=True)
        return carry

    lax.fori_loop(0, N_CH_D, body, 0)
    plsc.subcore_barrier()
    pltpu.sync_copy(acc.at[slab], out.at[c, slab])


BLK_P = 3128
GRID = P // BLK_P


def _tc_body(xh, agg, degp, bd, out):
    invd = 1.0 / (degp[0] + degp[1] + 1e-7)
    hl = 2.0 * xh[0] - agg[0] * invd
    hr = 2.0 * xh[1] - agg[1] * invd
    yl = jax.nn.sigmoid(
        jnp.dot(hl, bd[0, 0], preferred_element_type=jnp.float32)
        + jnp.dot(hr, bd[1, 0], preferred_element_type=jnp.float32))
    yr = jax.nn.sigmoid(
        jnp.dot(hl, bd[0, 1], preferred_element_type=jnp.float32)
        + jnp.dot(hr, bd[1, 1], preferred_element_type=jnp.float32))
    out[0] = yl
    out[1] = yr


def _tc_dense(xh, agg, degp, bd):
    spec3 = pl.BlockSpec((NC, BLK_P, 128), lambda i: (0, i, 0))
    return pl.pallas_call(
        _tc_body,
        out_shape=jax.ShapeDtypeStruct((NC, P, 128), jnp.float32),
        grid=(GRID,),
        in_specs=[spec3, spec3, spec3,
                  pl.BlockSpec((2, 2, 128, 128), lambda i: (0, 0, 0, 0))],
        out_specs=spec3,
    )(xh, agg, degp, bd)


def kernel(Gu, Gi, filters, edge_index):
    rowi = edge_index[0]
    coli = edge_index[1]
    x0 = jnp.concatenate(
        [Gu, Gi, jnp.zeros((NNP - NN, K), jnp.float32)], axis=0)
    yp = jnp.stack([x0[:, :KH].reshape(P, 128),
                    x0[:, KH:].reshape(P, 128)])

    eye8 = jnp.eye(8, dtype=jnp.float32)
    bds = jnp.stack([
        jnp.stack([
            jnp.stack([jnp.kron(eye8, filters[k][:KH, :KH]),
                       jnp.kron(eye8, filters[k][:KH, KH:])]),
            jnp.stack([jnp.kron(eye8, filters[k][KH:, :KH]),
                       jnp.kron(eye8, filters[k][KH:, KH:])]),
        ]) for k in range(NL)])

    degp = _sc_deg(rowi).reshape(NC, P, 128)

    pieces = [yp]
    for k in range(NL):
        agg = _sc_spmm(yp.reshape(NC, NNP, KH), coli,
                       rowi).reshape(NC, P, 128)
        yp = _tc_dense(yp, agg, degp, bds[k])
        pieces.append(yp)

    emb = jnp.concatenate(
        [p[h].reshape(NNP, KH) for p in pieces for h in (0, 1)], axis=1)
    return emb[:NU], emb[NU:NN]


# speedup vs baseline: 33.1037x; 1.1240x over previous
import functools

import jax
import jax.numpy as jnp
from jax import lax
from jax.experimental import pallas as pl
from jax.experimental.pallas import tpu as pltpu
from jax.experimental.pallas import tpu_sc as plsc

NU = 50000
NI = 50000
NN = NU + NI
NNP = 100096
P = NNP // 8
EDG = 1600000
K = 32
KH = 16
NL = 3

NC = 2
NS = 16

CH = 800
PER_TEC = EDG // NS
N_CH = PER_TEC // CH
ROWS_PER_TEC = NNP // NS

CH_D = 1000
PER_TEC_D = EDG // (NC * NS)
N_CH_D = PER_TEC_D // CH_D

_mesh = plsc.VectorSubcoreMesh(core_axis_name="c", subcore_axis_name="s")


@functools.partial(
    pl.kernel,
    out_type=jax.ShapeDtypeStruct((NC, NNP, KH), jnp.float32),
    mesh=_mesh,
    scratch_types=[
        pltpu.VMEM((3, CH), jnp.int32),
        pltpu.VMEM((3, CH), jnp.int32),
        pltpu.VMEM((2, CH, KH), jnp.float32),
        pltpu.VMEM_SHARED((NNP, KH), jnp.float32),
        pltpu.SemaphoreType.DMA,
        pltpu.SemaphoreType.DMA((2,)),
        pltpu.SemaphoreType.DMA,
    ],
    compiler_params=pltpu.CompilerParams(use_tc_tiling_on_sc=False),
)
def _sc_spmm(xcat, coli, rowi, out, colv, rowv, gat, acc, semi, semg, sems):
    c = lax.axis_index("c")
    s = lax.axis_index("s")
    slab = pl.ds(s * ROWS_PER_TEC, ROWS_PER_TEC)

    def fill_zero(j, carry):
        gat[0, j, :] = jnp.zeros((16,), jnp.float32)
        return carry

    lax.fori_loop(0, CH, fill_zero, 0)
    for r in range(7):
        pltpu.sync_copy(
            gat.at[0],
            acc.at[pl.ds(s * ROWS_PER_TEC + r * CH, CH)])
    pltpu.sync_copy(
        gat.at[0, pl.ds(0, 656)],
        acc.at[pl.ds(s * ROWS_PER_TEC + 7 * CH, 656)])
    plsc.subcore_barrier()

    base0 = s * PER_TEC

    def start_idx(g, b):
        pltpu.async_copy(coli.at[pl.ds(base0 + g * CH, CH)], colv.at[b], semi)
        pltpu.async_copy(rowi.at[pl.ds(base0 + g * CH, CH)], rowv.at[b], semi)

    def wait_idx(b):
        pltpu.make_async_copy(coli.at[pl.ds(base0, CH)], colv.at[b], semi).wait()
        pltpu.make_async_copy(rowi.at[pl.ds(base0, CH)], rowv.at[b], semi).wait()

    def start_gather(b3, b2):
        pltpu.async_copy(xcat.at[c].at[colv.at[b3]], gat.at[b2], semg.at[b2])

    def wait_gather(b3, b2):
        pltpu.make_async_copy(xcat.at[c].at[colv.at[b3]], gat.at[b2],
                              semg.at[b2]).wait()

    def start_scatter(b2, b3):
        pltpu.async_copy(gat.at[b2], acc.at[rowv.at[b3]], sems, add=True)

    def wait_scatter(b2, b3):
        pltpu.make_async_copy(gat.at[b2], acc.at[rowv.at[b3]], sems).wait()

    start_idx(0, 0)
    wait_idx(0)
    start_idx(1, 1)
    start_gather(0, 0)

    def body(g, carry):
        pg = lax.rem(g, 2)
        qg = 1 - pg
        b0 = lax.rem(g, 3)
        b1 = lax.rem(g + 1, 3)
        b2 = lax.rem(g + 2, 3)

        @pl.when(g >= 1)
        def _():
            wait_scatter(qg, b2)

        @pl.when(g + 1 < N_CH)
        def _():
            wait_idx(b1)
            start_gather(b1, qg)

        wait_gather(b0, pg)
        start_scatter(pg, b0)

        @pl.when(g + 2 < N_CH)
        def _():
            start_idx(g + 2, b2)

        return carry

    lax.fori_loop(0, N_CH, body, 0)
    wait_scatter(lax.rem(N_CH - 1, 2), lax.rem(N_CH - 1, 3))
    plsc.subcore_barrier()
    pltpu.sync_copy(acc.at[slab], out.at[c, slab])


@functools.partial(
    pl.kernel,
    out_type=jax.ShapeDtypeStruct((NC, NNP, KH), jnp.float32),
    mesh=_mesh,
    scratch_types=[
        pltpu.VMEM((CH_D,), jnp.int32),
        pltpu.VMEM((CH_D, KH), jnp.float32),
        pltpu.VMEM_SHARED((NNP, KH), jnp.float32),
    ],
    compiler_params=pltpu.CompilerParams(use_tc_tiling_on_sc=False),
)
def _sc_deg(rowi, out, rowv, onev, acc):
    c = lax.axis_index("c")
    s = lax.axis_index("s")
    slab = pl.ds(s * ROWS_PER_TEC, ROWS_PER_TEC)

    def fill(j, val):
        onev[j, :] = jnp.full((16,), val, jnp.float32)
        return val

    lax.fori_loop(0, CH_D, lambda j, v: fill(j, 0.0), 0.0)
    for r in range(6):
        pltpu.sync_copy(
            onev, acc.at[pl.ds(s * ROWS_PER_TEC + r * CH_D, CH_D)])
    pltpu.sync_copy(
        onev.at[pl.ds(0, 256)],
        acc.at[pl.ds(s * ROWS_PER_TEC + 6 * CH_D, 256)])
    lax.fori_loop(0, CH_D, lambda j, v: fill(j, 1.0), 1.0)
    plsc.subcore_barrier()

    def body(g, carry):
        base = (c * NS + s) * PER_TEC_D + g * CH_D
        pltpu.sync_copy(rowi.at[pl.ds(base, CH_D)], rowv)
        pltpu.sync_copy(onev, acc.at[rowv], add=True)
        return carry

    lax.fori_loop(0, N_CH_D, body, 0)
    plsc.subcore_barrier()
    pltpu.sync_copy(acc.at[slab], out.at[c, slab])


BLK_P = 3128
GRID = P // BLK_P


def _tc_body(xh, agg, degp, bd, out):
    invd = 1.0 / (degp[0] + degp[1] + 1e-7)
    hl = 2.0 * xh[0] - agg[0] * invd
    hr = 2.0 * xh[1] - agg[1] * invd
    yl = jax.nn.sigmoid(
        jnp.dot(hl, bd[0, 0], preferred_element_type=jnp.float32)
        + jnp.dot(hr, bd[1, 0], preferred_element_type=jnp.float32))
    yr = jax.nn.sigmoid(
        jnp.dot(hl, bd[0, 1], preferred_element_type=jnp.float32)
        + jnp.dot(hr, bd[1, 1], preferred_element_type=jnp.float32))
    out[0] = yl
    out[1] = yr


def _tc_dense(xh, agg, degp, bd):
    spec3 = pl.BlockSpec((NC, BLK_P, 128), lambda i: (0, i, 0))
    return pl.pallas_call(
        _tc_body,
        out_shape=jax.ShapeDtypeStruct((NC, P, 128), jnp.float32),
        grid=(GRID,),
        in_specs=[spec3, spec3, spec3,
                  pl.BlockSpec((2, 2, 128, 128), lambda i: (0, 0, 0, 0))],
        out_specs=spec3,
    )(xh, agg, degp, bd)


def kernel(Gu, Gi, filters, edge_index):
    rowi = edge_index[0]
    coli = edge_index[1]
    x0 = jnp.concatenate(
        [Gu, Gi, jnp.zeros((NNP - NN, K), jnp.float32)], axis=0)
    yp = jnp.stack([x0[:, :KH].reshape(P, 128),
                    x0[:, KH:].reshape(P, 128)])

    eye8 = jnp.eye(8, dtype=jnp.float32)
    bds = jnp.stack([
        jnp.stack([
            jnp.stack([jnp.kron(eye8, filters[k][:KH, :KH]),
                       jnp.kron(eye8, filters[k][:KH, KH:])]),
            jnp.stack([jnp.kron(eye8, filters[k][KH:, :KH]),
                       jnp.kron(eye8, filters[k][KH:, KH:])]),
        ]) for k in range(NL)])

    degp = _sc_deg(rowi).reshape(NC, P, 128)

    pieces = [yp]
    for k in range(NL):
        agg = _sc_spmm(yp.reshape(NC, NNP, KH), coli,
                       rowi).reshape(NC, P, 128)
        yp = _tc_dense(yp, agg, degp, bds[k])
        pieces.append(yp)

    emb = jnp.concatenate(
        [p[h].reshape(NNP, KH) for p in pieces for h in (0, 1)], axis=1)
    return emb[:NU], emb[NU:NN]
